# baseline (speedup 1.0000x reference)
import functools

import jax
import jax.numpy as jnp
import numpy as np
from jax import lax
from jax.experimental import pallas as pl
from jax.experimental.pallas import tpu as pltpu
from jax.experimental.pallas import tpu_sc as plsc

DIM_ = 128
B_ = 16384
NC_ = 2
NS_ = 16
NW_ = NC_ * NS_
NCHUNK_ = 4
CB_ = B_ // NCHUNK_
BPW_ = CB_ // NW_
BM_ = 1024
NB_ = CB_ // BM_

_PERM = np.empty((DIM_,), np.int64)
for _k in range(DIM_ // 32):
    for _j in range(16):
        _PERM[32 * _k + 2 * _j] = 32 * _k + _j
        _PERM[32 * _k + 2 * _j + 1] = 32 * _k + 16 + _j
_PMAT = np.zeros((DIM_, DIM_), np.float32)
for _i in range(DIM_):
    _PMAT[_i, _PERM[_i]] = 1.0


def _preproj_body(ed_ref, et_ref, w_ref, b_ref, o1_ref, o2_ref):
    o1_ref[...] = (jnp.dot(ed_ref[...], w_ref[pl.ds(DIM_, DIM_), :],
                           preferred_element_type=jnp.float32)
                   + b_ref[...])
    o2_ref[...] = jnp.dot(et_ref[...], w_ref[pl.ds(2 * DIM_, DIM_), :],
                          preferred_element_type=jnp.float32)


def _preproj(e_dose, e_time, w, b2):
    n = e_dose.shape[0]
    sds = jax.ShapeDtypeStruct((n, DIM_), jnp.float32)
    return pl.pallas_call(
        _preproj_body,
        out_shape=[sds, sds],
    )(e_dose, e_time, w, b2)


_RND = jnp.uint32(0x8000)
_HI = jnp.uint32(0xFFFF0000)


def _to_bf16_pair(a32, b32):
    au = plsc.bitcast(a32, jnp.uint32)
    bu = plsc.bitcast(b32, jnp.uint32)
    au = lax.shift_right_logical(au + _RND, jnp.uint32(16))
    bu = (bu + _RND) & _HI
    return au | bu


def _cvt_rows(src32, dstbf, nrows):
    def _row(r, carry):
        for k in range(DIM_ // 32):
            dstbf[r, pl.ds(k * 16, 16)] = _to_bf16_pair(
                src32[r, pl.ds(k * 32, 16)],
                src32[r, pl.ds(k * 32 + 16, 16)])
        return carry

    lax.fori_loop(0, nrows, _row, 0, unroll=2)


def _cvt_sum_rows(s1, s2, dstbf, nrows):
    def _row(r, carry):
        for k in range(DIM_ // 32):
            a = (s1[r, pl.ds(k * 32, 16)] + s2[r, pl.ds(k * 32, 16)])
            bb = (s1[r, pl.ds(k * 32 + 16, 16)]
                  + s2[r, pl.ds(k * 32 + 16, 16)])
            dstbf[r, pl.ds(k * 16, 16)] = _to_bf16_pair(a, bb)
        return carry

    lax.fori_loop(0, nrows, _row, 0, unroll=2)


def _sc_gather_body(c, ic_hbm, id_hbm, it_hbm, ib_hbm, tc_hbm, tb_hbm,
                    p1_hbm, p2_hbm, obig_hbm, os_hbm,
                    idx_v, rows_v, s1_v, s2_v, bf_v, sbf_v,
                    isem, gsem, wsem):
    wid = lax.axis_index("s") * NC_ + lax.axis_index("c")
    base = wid * BPW_
    src = c * CB_ + base
    ics = [
        pltpu.async_copy(h.at[pl.ds(src, BPW_)], idx_v.at[t], isem)
        for t, h in enumerate((id_hbm, it_hbm, ic_hbm, ib_hbm))
    ]
    for ic in ics:
        ic.wait()
    g1 = pltpu.async_copy(p1_hbm.at[idx_v.at[0]], s1_v, gsem)
    g2 = pltpu.async_copy(p2_hbm.at[idx_v.at[1]], s2_v, gsem)
    g0 = pltpu.async_copy(tc_hbm.at[idx_v.at[2]], rows_v.at[0], gsem)
    g3 = pltpu.async_copy(tb_hbm.at[idx_v.at[3]], rows_v.at[1], gsem)
    g1.wait()
    g2.wait()
    _cvt_sum_rows(s1_v, s2_v, sbf_v, BPW_)
    ws = pltpu.async_copy(sbf_v, os_hbm.at[pl.ds(base, BPW_)], wsem)
    g0.wait()
    _cvt_rows(rows_v.at[0], bf_v.at[0], BPW_)
    w0 = pltpu.async_copy(bf_v.at[0], obig_hbm.at[0, pl.ds(base, BPW_)],
                          wsem)
    g3.wait()
    _cvt_rows(rows_v.at[1], bf_v.at[1], BPW_)
    w1 = pltpu.async_copy(bf_v.at[1], obig_hbm.at[1, pl.ds(base, BPW_)],
                          wsem)
    ws.wait()
    w0.wait()
    w1.wait()


def _make_gather(c):
    return pl.kernel(
        functools.partial(_sc_gather_body, c),
        out_type=[
            jax.ShapeDtypeStruct((2, CB_, DIM_ // 2), jnp.uint32),
            jax.ShapeDtypeStruct((CB_, DIM_ // 2), jnp.uint32),
        ],
        mesh=plsc.VectorSubcoreMesh(core_axis_name="c",
                                    subcore_axis_name="s"),
        compiler_params=pltpu.CompilerParams(needs_layout_passes=False),
        scratch_types=[
            pltpu.VMEM((4, BPW_), jnp.int32),
            pltpu.VMEM((2, BPW_, DIM_), jnp.float32),
            pltpu.VMEM((BPW_, DIM_), jnp.float32),
            pltpu.VMEM((BPW_, DIM_), jnp.float32),
            pltpu.VMEM((2, BPW_, DIM_ // 2), jnp.uint32),
            pltpu.VMEM((BPW_, DIM_ // 2), jnp.uint32),
            pltpu.SemaphoreType.DMA,
            pltpu.SemaphoreType.DMA,
            pltpu.SemaphoreType.DMA,
        ],
    )


_gathers = [_make_gather(c) for c in range(NCHUNK_)]


def _proj_body(x_ref, s_ref, wp_ref, pm_ref, o_ref):
    o_ref[...] = (
        jnp.dot(s_ref[...], pm_ref[...], preferred_element_type=jnp.float32)
        + jnp.dot(x_ref[0], wp_ref[pl.ds(0, DIM_), :],
                  preferred_element_type=jnp.float32)
        + jnp.dot(x_ref[1], wp_ref[pl.ds(DIM_, DIM_), :],
                  preferred_element_type=jnp.float32))


def _proj_body_alias(x_ref, s_ref, wp_ref, pm_ref, buf_ref, o_ref):
    o_ref[...] = (
        jnp.dot(s_ref[...], pm_ref[...], preferred_element_type=jnp.float32)
        + jnp.dot(x_ref[0], wp_ref[pl.ds(0, DIM_), :],
                  preferred_element_type=jnp.float32)
        + jnp.dot(x_ref[1], wp_ref[pl.ds(DIM_, DIM_), :],
                  preferred_element_type=jnp.float32))


def _proj(c, x, s, wp2, pm, buf):
    common = dict(
        grid=(NB_,),
        out_specs=pl.BlockSpec((BM_, DIM_), lambda i: (c * NB_ + i, 0)),
        out_shape=jax.ShapeDtypeStruct((B_, DIM_), jnp.float32),
    )
    in_specs = [
        pl.BlockSpec((2, BM_, DIM_), lambda i: (0, i, 0)),
        pl.BlockSpec((BM_, DIM_), lambda i: (i, 0)),
        pl.BlockSpec((2 * DIM_, DIM_), lambda i: (0, 0)),
        pl.BlockSpec((DIM_, DIM_), lambda i: (0, 0)),
    ]
    if buf is None:
        return pl.pallas_call(
            _proj_body,
            in_specs=in_specs,
            **common,
        )(x, s, wp2, pm)
    return pl.pallas_call(
        _proj_body_alias,
        in_specs=in_specs + [pl.BlockSpec(memory_space=pl.ANY)],
        input_output_aliases={4: 0},
        **common,
    )(x, s, wp2, pm, buf)


def _as_bf16(u32_arr):
    bf = lax.bitcast_convert_type(u32_arr, jnp.bfloat16)
    return bf.reshape(*u32_arr.shape[:-1], u32_arr.shape[-1] * 2)


def kernel(cell_type, dose, time, batch, E_cell_type, E_dose, E_time,
           E_batch, W, b):
    ic = cell_type.astype(jnp.int32)
    id_ = dose.astype(jnp.int32)
    it = time.astype(jnp.int32)
    ib = batch.astype(jnp.int32)
    b2 = b.reshape(1, DIM_)
    perm = jnp.asarray(_PERM)
    w_perm = W.reshape(4, DIM_, DIM_)[:, perm, :].reshape(4 * DIM_, DIM_)
    wp2 = jnp.concatenate([w_perm[:DIM_], w_perm[3 * DIM_:]], axis=0)
    pm = jnp.asarray(_PMAT)
    p1, p2 = _preproj(E_dose, E_time, W, b2)
    buf = None
    for c in range(NCHUNK_):
        xbf, sbf = _gathers[c](ic, id_, it, ib, E_cell_type, E_batch,
                               p1, p2)
        buf = _proj(c, _as_bf16(xbf), _as_bf16(sbf), wp2, pm, buf)
    return buf
